# Initial kernel scaffold; baseline (speedup 1.0000x reference)
import jax
import jax.numpy as jnp
from jax.experimental import pallas as pl


def kernel(x_user, edge_index_follow, edge_index_friend, W_num, b_num, W_cat, b_cat, W_des, b_des, W_tweet, b_tweet, W_in, b_in, W_f0, b_f0, W_r0, b_r0, W_f1, b_f1, W_r1, b_r1):
    raise NotImplementedError("write your pallas kernel here")


# speedup vs baseline: 7.2159x; 7.2159x over previous
import functools

import jax
import jax.numpy as jnp
from jax import lax
from jax.experimental import pallas as pl
from jax.experimental.pallas import tpu as pltpu
from jax.experimental.pallas import tpu_sc as plsc

N = 10000
E = 160000
D = 128
Q = 32
NUM_PROP = 5
CAT_PROP = 3
DES = 768
TWEET = 768
KX = NUM_PROP + CAT_PROP + DES + TWEET

NC = 2
NS = 16
B = 128
CH = 80
NH = 2
HCH = CH // NH
EPAD = NS * CH * B
NPAD = N + 8
R0 = 624
TAIL = N - NS * R0
TB = N - TAIL


@functools.cache
def _sc_mesh():
    return plsc.VectorSubcoreMesh(
        core_axis_name="c", subcore_axis_name="s", num_cores=NC, num_subcores=NS
    )


def _sc_degree_body(ones_hbm, dst_hbm, out_hbm, dst_v, ones_v, acc):
    c = lax.axis_index("c")
    s = lax.axis_index("s")
    pltpu.sync_copy(ones_hbm, acc.at[pl.ds(s * R0, R0)])

    @pl.when(s == NS - 1)
    def _():
        pltpu.sync_copy(ones_hbm.at[pl.ds(0, TAIL)], acc.at[pl.ds(TB, TAIL)])

    pltpu.sync_copy(ones_hbm.at[pl.ds(0, B)], ones_v)
    plsc.subcore_barrier()

    def body(j, carry):
        pltpu.sync_copy(ones_v, acc.at[dst_v.at[j]], add=True)
        return carry

    for h in range(NH):
        pltpu.sync_copy(dst_hbm.at[c, s, h], dst_v)
        lax.fori_loop(0, HCH, body, 0)
    plsc.subcore_barrier()
    pltpu.sync_copy(
        acc.at[pl.ds(s * R0, R0)], out_hbm.at[c, pl.ds(s * R0, R0)])

    @pl.when(s == NS - 1)
    def _():
        pltpu.sync_copy(acc.at[pl.ds(TB, TAIL)], out_hbm.at[c, pl.ds(TB, TAIL)])


@functools.cache
def _sc_degree():
    return pl.kernel(
        _sc_degree_body,
        out_type=jax.ShapeDtypeStruct((NC, N, D), jnp.float32),
        mesh=_sc_mesh(),
        scratch_types=[
            pltpu.VMEM((HCH, B), jnp.int32),
            pltpu.VMEM((B, D), jnp.float32),
            pltpu.VMEM_SHARED((NPAD, D), jnp.float32),
        ],
    )


def _sc_aggregate_body(h_hbm, src_hbm, dst_hbm, out_hbm,
                       src_v, dst_v, rows0, rows1, acc, sem0, sem1):
    c = lax.axis_index("c")
    s = lax.axis_index("s")
    pltpu.sync_copy(
        h_hbm.at[pl.ds(c * N + s * R0, R0)], acc.at[pl.ds(s * R0, R0)])

    @pl.when(s == NS - 1)
    def _():
        pltpu.sync_copy(h_hbm.at[pl.ds(c * N + TB, TAIL)], acc.at[pl.ds(TB, TAIL)])

    plsc.subcore_barrier()

    def body(g, carry):
        j = 2 * g
        pltpu.make_async_copy(h_hbm.at[src_v.at[j]], rows0, sem0).wait()
        pltpu.async_copy(h_hbm.at[src_v.at[j + 1]], rows1, sem1)
        pltpu.sync_copy(rows0, acc.at[dst_v.at[j]], add=True)
        pltpu.make_async_copy(h_hbm.at[src_v.at[j + 1]], rows1, sem1).wait()
        pltpu.async_copy(h_hbm.at[src_v.at[j + 2]], rows0, sem0)
        pltpu.sync_copy(rows1, acc.at[dst_v.at[j + 1]], add=True)
        return carry

    for h in range(NH):
        pltpu.sync_copy(src_hbm.at[c, s, h], src_v)
        pltpu.sync_copy(dst_hbm.at[c, s, h], dst_v)
        pltpu.async_copy(h_hbm.at[src_v.at[0]], rows0, sem0)
        lax.fori_loop(0, HCH // 2, body, 0)
        pltpu.make_async_copy(h_hbm.at[src_v.at[HCH]], rows0, sem0).wait()
    plsc.subcore_barrier()
    pltpu.sync_copy(
        acc.at[pl.ds(s * R0, R0)], out_hbm.at[c, pl.ds(s * R0, R0)])

    @pl.when(s == NS - 1)
    def _():
        pltpu.sync_copy(acc.at[pl.ds(TB, TAIL)], out_hbm.at[c, pl.ds(TB, TAIL)])


@functools.cache
def _sc_aggregate():
    return pl.kernel(
        _sc_aggregate_body,
        out_type=jax.ShapeDtypeStruct((NC, N, D), jnp.float32),
        mesh=_sc_mesh(),
        scratch_types=[
            pltpu.VMEM((HCH + 1, B), jnp.int32),
            pltpu.VMEM((HCH, B), jnp.int32),
            pltpu.VMEM((B, D), jnp.float32),
            pltpu.VMEM((B, D), jnp.float32),
            pltpu.VMEM_SHARED((NPAD, D), jnp.float32),
            pltpu.SemaphoreType.DMA,
            pltpu.SemaphoreType.DMA,
        ],
    )


def _lrelu(v):
    return jnp.where(v >= 0, v, 0.01 * v)


def _dot(a, b):
    return jnp.dot(a, b, preferred_element_type=jnp.float32,
                   precision=lax.Precision.HIGHEST)


BM = 400


def _tc_encode_body(x_ref, w1_ref, b1_ref, win_ref, bin_ref, cf_ref, cr_ref,
                    wf_ref, wr_ref, hs_ref, df_ref, dr_ref):
    feat = _lrelu(_dot(x_ref[...], w1_ref[...]) + b1_ref[...])
    emb = _lrelu(_dot(feat, win_ref[...]) + bin_ref[...])
    df = lax.rsqrt(cf_ref[...])
    dr = lax.rsqrt(cr_ref[...])
    df_ref[...] = df
    dr_ref[...] = dr
    hs_ref[...] = jnp.stack(
        [_dot(emb, wf_ref[...]) * df, _dot(emb, wr_ref[...]) * dr])


def _tc_mid_body(a_ref, df_ref, dr_ref, bf_ref, br_ref,
                 wf_ref, wr_ref, hs_ref):
    h1 = jnp.maximum(
        df_ref[...] * a_ref[0] + bf_ref[...]
        + dr_ref[...] * a_ref[1] + br_ref[...], 0.0)
    hs_ref[...] = jnp.stack(
        [_dot(h1, wf_ref[...]) * df_ref[...], _dot(h1, wr_ref[...]) * dr_ref[...]])


def _tc_out_body(a_ref, df_ref, dr_ref, bf_ref, br_ref, out_ref):
    out_ref[...] = jnp.maximum(
        df_ref[...] * a_ref[0] + bf_ref[...]
        + dr_ref[...] * a_ref[1] + br_ref[...], 0.0)


def _row_spec(w):
    return pl.BlockSpec((BM, w), lambda i: (i, 0))


def _stk_spec():
    return pl.BlockSpec((NC, BM, D), lambda i: (0, i, 0))


def _full_spec(h, w):
    return pl.BlockSpec((h, w), lambda i: (0, 0))


_HS = jax.ShapeDtypeStruct((NC, N, D), jnp.float32)

_tc_encode = pl.pallas_call(
    _tc_encode_body,
    grid=(N // BM,),
    in_specs=[
        _row_spec(KX),
        _full_spec(KX, D), _full_spec(1, D),
        _full_spec(D, D), _full_spec(1, D),
        _row_spec(1), _row_spec(1),
        _full_spec(D, D), _full_spec(D, D),
    ],
    out_specs=[_stk_spec(), _row_spec(1), _row_spec(1)],
    out_shape=[
        _HS,
        jax.ShapeDtypeStruct((N, 1), jnp.float32),
        jax.ShapeDtypeStruct((N, 1), jnp.float32),
    ],
)

_tc_mid = pl.pallas_call(
    _tc_mid_body,
    grid=(N // BM,),
    in_specs=[
        _stk_spec(), _row_spec(1), _row_spec(1),
        _full_spec(1, D), _full_spec(1, D),
        _full_spec(D, D), _full_spec(D, D),
    ],
    out_specs=_stk_spec(),
    out_shape=_HS,
)

_tc_out = pl.pallas_call(
    _tc_out_body,
    grid=(N // BM,),
    in_specs=[
        _stk_spec(), _row_spec(1), _row_spec(1),
        _full_spec(1, D), _full_spec(1, D),
    ],
    out_specs=_row_spec(D),
    out_shape=jax.ShapeDtypeStruct((N, D), jnp.float32),
)


def kernel(x_user, edge_index_follow, edge_index_friend,
           W_num, b_num, W_cat, b_cat, W_des, b_des, W_tweet, b_tweet,
           W_in, b_in, W_f0, b_f0, W_r0, b_r0, W_f1, b_f1, W_r1, b_r1):
    W1 = jnp.zeros((KX, D), jnp.float32)
    W1 = W1.at[:NUM_PROP, :Q].set(W_num)
    W1 = W1.at[NUM_PROP:NUM_PROP + CAT_PROP, Q:2 * Q].set(W_cat)
    W1 = W1.at[NUM_PROP + CAT_PROP:NUM_PROP + CAT_PROP + DES, 2 * Q:3 * Q].set(W_des)
    W1 = W1.at[NUM_PROP + CAT_PROP + DES:, 3 * Q:].set(W_tweet)
    b1 = jnp.concatenate([b_num, b_cat, b_des, b_tweet]).reshape(1, D)

    npad = EPAD - E
    pad8 = jnp.arange(npad, dtype=jnp.int32) % 8
    zchunk = jnp.zeros((NS, NH, 1, B), jnp.int32)

    def prep(ei, c):
        src = jnp.concatenate([ei[0] + c * N, pad8]).reshape(NS, NH, HCH, B)
        src = jnp.concatenate([src, zchunk], axis=2)
        dst = jnp.concatenate([ei[1], N + pad8]).reshape(NS, NH, HCH, B)
        return src, dst

    s_f, d_f = prep(edge_index_follow, 0)
    s_r, d_r = prep(edge_index_friend, 1)
    srcI = jnp.stack([s_f, s_r])
    dstI = jnp.stack([d_f, d_r])

    ones_r = jnp.ones((R0, D), jnp.float32)
    deg = _sc_degree()(ones_r, dstI)
    cnt_f = deg[0, :, 0:1]
    cnt_r = deg[1, :, 0:1]

    hs0, dinv_f, dinv_r = _tc_encode(
        x_user, W1, b1, W_in, b_in.reshape(1, D), cnt_f, cnt_r, W_f0, W_r0)

    agg0 = _sc_aggregate()(hs0.reshape(NC * N, D), srcI, dstI)

    hs1 = _tc_mid(agg0, dinv_f, dinv_r,
                  b_f0.reshape(1, D), b_r0.reshape(1, D), W_f1, W_r1)

    agg1 = _sc_aggregate()(hs1.reshape(NC * N, D), srcI, dstI)

    return _tc_out(agg1, dinv_f, dinv_r,
                   b_f1.reshape(1, D), b_r1.reshape(1, D))


# speedup vs baseline: 13.3529x; 1.8505x over previous
import functools

import jax
import jax.numpy as jnp
from jax import lax
from jax.experimental import pallas as pl
from jax.experimental.pallas import tpu as pltpu
from jax.experimental.pallas import tpu_sc as plsc

N = 10000
E = 160000
D = 128
Q = 32
NUM_PROP = 5
CAT_PROP = 3
DES = 768
TWEET = 768
KX = NUM_PROP + CAT_PROP + DES + TWEET

NC = 2
NS = 16
B = 128
CH = 80
NH = 2
HCH = CH // NH
EPAD = NS * CH * B
NPAD = N + 8
R0 = 624
TAIL = N - NS * R0
TB = N - TAIL


@functools.cache
def _sc_mesh():
    return plsc.VectorSubcoreMesh(
        core_axis_name="c", subcore_axis_name="s", num_cores=NC, num_subcores=NS
    )


def _sc_degree_body(ones_hbm, dst_hbm, out_hbm, dst_v, ones_v, acc):
    c = lax.axis_index("c")
    s = lax.axis_index("s")
    pltpu.sync_copy(ones_hbm, acc.at[pl.ds(s * R0, R0)])

    @pl.when(s == NS - 1)
    def _():
        pltpu.sync_copy(ones_hbm.at[pl.ds(0, TAIL)], acc.at[pl.ds(TB, TAIL)])

    pltpu.sync_copy(ones_hbm.at[pl.ds(0, B)], ones_v)
    plsc.subcore_barrier()

    def body(j, carry):
        pltpu.sync_copy(ones_v, acc.at[dst_v.at[j]], add=True)
        return carry

    for h in range(NH):
        pltpu.sync_copy(dst_hbm.at[c, s, h], dst_v)
        lax.fori_loop(0, HCH, body, 0)
    plsc.subcore_barrier()
    pltpu.sync_copy(
        acc.at[pl.ds(s * R0, R0)], out_hbm.at[c, pl.ds(s * R0, R0)])

    @pl.when(s == NS - 1)
    def _():
        pltpu.sync_copy(acc.at[pl.ds(TB, TAIL)], out_hbm.at[c, pl.ds(TB, TAIL)])


@functools.cache
def _sc_degree():
    return pl.kernel(
        _sc_degree_body,
        out_type=jax.ShapeDtypeStruct((NC, N, D), jnp.float32),
        mesh=_sc_mesh(),
        scratch_types=[
            pltpu.VMEM((HCH, B), jnp.int32),
            pltpu.VMEM((B, D), jnp.float32),
            pltpu.VMEM_SHARED((NPAD, D), jnp.float32),
        ],
    )


def _sc_aggregate_body(h_hbm, src_hbm, dst_hbm, out_hbm,
                       src_v, dst_v, rows0, rows1, acc, sem0, sem1):
    c = lax.axis_index("c")
    s = lax.axis_index("s")
    pltpu.sync_copy(
        h_hbm.at[pl.ds(c * N + s * R0, R0)], acc.at[pl.ds(s * R0, R0)])

    @pl.when(s == NS - 1)
    def _():
        pltpu.sync_copy(h_hbm.at[pl.ds(c * N + TB, TAIL)], acc.at[pl.ds(TB, TAIL)])

    plsc.subcore_barrier()

    def body(j, carry):
        pltpu.sync_copy(h_hbm.at[src_v.at[j]], rows0)
        pltpu.sync_copy(rows0, acc.at[dst_v.at[j]], add=True)
        return carry

    for h in range(NH):
        pltpu.sync_copy(src_hbm.at[c, s, h], src_v)
        pltpu.sync_copy(dst_hbm.at[c, s, h], dst_v)
        lax.fori_loop(0, HCH, body, 0)
    plsc.subcore_barrier()
    pltpu.sync_copy(
        acc.at[pl.ds(s * R0, R0)], out_hbm.at[c, pl.ds(s * R0, R0)])

    @pl.when(s == NS - 1)
    def _():
        pltpu.sync_copy(acc.at[pl.ds(TB, TAIL)], out_hbm.at[c, pl.ds(TB, TAIL)])


@functools.cache
def _sc_aggregate():
    return pl.kernel(
        _sc_aggregate_body,
        out_type=jax.ShapeDtypeStruct((NC, N, D), jnp.float32),
        mesh=_sc_mesh(),
        scratch_types=[
            pltpu.VMEM((HCH + 1, B), jnp.int32),
            pltpu.VMEM((HCH, B), jnp.int32),
            pltpu.VMEM((B, D), jnp.float32),
            pltpu.VMEM((B, D), jnp.float32),
            pltpu.VMEM_SHARED((NPAD, D), jnp.float32),
            pltpu.SemaphoreType.DMA,
            pltpu.SemaphoreType.DMA,
        ],
    )


def _lrelu(v):
    return jnp.where(v >= 0, v, 0.01 * v)


def _dot(a, b):
    return jnp.dot(a, b, preferred_element_type=jnp.float32,
                   precision=lax.Precision.HIGHEST)


BM = 400


def _tc_encode_body(x_ref, w1_ref, b1_ref, win_ref, bin_ref, cf_ref, cr_ref,
                    wf_ref, wr_ref, hs_ref, df_ref, dr_ref):
    feat = _lrelu(_dot(x_ref[...], w1_ref[...]) + b1_ref[...])
    emb = _lrelu(_dot(feat, win_ref[...]) + bin_ref[...])
    df = lax.rsqrt(cf_ref[...])
    dr = lax.rsqrt(cr_ref[...])
    df_ref[...] = df
    dr_ref[...] = dr
    hs_ref[...] = jnp.stack(
        [_dot(emb, wf_ref[...]) * df, _dot(emb, wr_ref[...]) * dr])


def _tc_mid_body(a_ref, df_ref, dr_ref, bf_ref, br_ref,
                 wf_ref, wr_ref, hs_ref):
    h1 = jnp.maximum(
        df_ref[...] * a_ref[0] + bf_ref[...]
        + dr_ref[...] * a_ref[1] + br_ref[...], 0.0)
    hs_ref[...] = jnp.stack(
        [_dot(h1, wf_ref[...]) * df_ref[...], _dot(h1, wr_ref[...]) * dr_ref[...]])


def _tc_out_body(a_ref, df_ref, dr_ref, bf_ref, br_ref, out_ref):
    out_ref[...] = jnp.maximum(
        df_ref[...] * a_ref[0] + bf_ref[...]
        + dr_ref[...] * a_ref[1] + br_ref[...], 0.0)


def _row_spec(w):
    return pl.BlockSpec((BM, w), lambda i: (i, 0))


def _stk_spec():
    return pl.BlockSpec((NC, BM, D), lambda i: (0, i, 0))


def _full_spec(h, w):
    return pl.BlockSpec((h, w), lambda i: (0, 0))


_HS = jax.ShapeDtypeStruct((NC, N, D), jnp.float32)

_tc_encode = pl.pallas_call(
    _tc_encode_body,
    grid=(N // BM,),
    in_specs=[
        _row_spec(KX),
        _full_spec(KX, D), _full_spec(1, D),
        _full_spec(D, D), _full_spec(1, D),
        _row_spec(1), _row_spec(1),
        _full_spec(D, D), _full_spec(D, D),
    ],
    out_specs=[_stk_spec(), _row_spec(1), _row_spec(1)],
    out_shape=[
        _HS,
        jax.ShapeDtypeStruct((N, 1), jnp.float32),
        jax.ShapeDtypeStruct((N, 1), jnp.float32),
    ],
)

_tc_mid = pl.pallas_call(
    _tc_mid_body,
    grid=(N // BM,),
    in_specs=[
        _stk_spec(), _row_spec(1), _row_spec(1),
        _full_spec(1, D), _full_spec(1, D),
        _full_spec(D, D), _full_spec(D, D),
    ],
    out_specs=_stk_spec(),
    out_shape=_HS,
)

_tc_out = pl.pallas_call(
    _tc_out_body,
    grid=(N // BM,),
    in_specs=[
        _stk_spec(), _row_spec(1), _row_spec(1),
        _full_spec(1, D), _full_spec(1, D),
    ],
    out_specs=_row_spec(D),
    out_shape=jax.ShapeDtypeStruct((N, D), jnp.float32),
)


def kernel(x_user, edge_index_follow, edge_index_friend,
           W_num, b_num, W_cat, b_cat, W_des, b_des, W_tweet, b_tweet,
           W_in, b_in, W_f0, b_f0, W_r0, b_r0, W_f1, b_f1, W_r1, b_r1):
    W1 = jnp.zeros((KX, D), jnp.float32)
    W1 = W1.at[:NUM_PROP, :Q].set(W_num)
    W1 = W1.at[NUM_PROP:NUM_PROP + CAT_PROP, Q:2 * Q].set(W_cat)
    W1 = W1.at[NUM_PROP + CAT_PROP:NUM_PROP + CAT_PROP + DES, 2 * Q:3 * Q].set(W_des)
    W1 = W1.at[NUM_PROP + CAT_PROP + DES:, 3 * Q:].set(W_tweet)
    b1 = jnp.concatenate([b_num, b_cat, b_des, b_tweet]).reshape(1, D)

    npad = EPAD - E
    pad8 = jnp.arange(npad, dtype=jnp.int32) % 8
    zchunk = jnp.zeros((NS, NH, 1, B), jnp.int32)

    def prep(ei, c):
        src = jnp.concatenate([ei[0] + c * N, pad8]).reshape(NS, NH, HCH, B)
        src = jnp.concatenate([src, zchunk], axis=2)
        dst = jnp.concatenate([ei[1], N + pad8]).reshape(NS, NH, HCH, B)
        return src, dst

    s_f, d_f = prep(edge_index_follow, 0)
    s_r, d_r = prep(edge_index_friend, 1)
    srcI = jnp.stack([s_f, s_r])
    dstI = jnp.stack([d_f, d_r])

    ones_r = jnp.ones((R0, D), jnp.float32)
    deg = _sc_degree()(ones_r, dstI)
    cnt_f = deg[0, :, 0:1]
    cnt_r = deg[1, :, 0:1]

    hs0, dinv_f, dinv_r = _tc_encode(
        x_user, W1, b1, W_in, b_in.reshape(1, D), cnt_f, cnt_r, W_f0, W_r0)

    agg0 = _sc_aggregate()(hs0.reshape(NC * N, D), srcI, dstI)

    hs1 = _tc_mid(agg0, dinv_f, dinv_r,
                  b_f0.reshape(1, D), b_r0.reshape(1, D), W_f1, W_r1)

    agg1 = _sc_aggregate()(hs1.reshape(NC * N, D), srcI, dstI)

    return _tc_out(agg1, dinv_f, dinv_r,
                   b_f1.reshape(1, D), b_r1.reshape(1, D))


# speedup vs baseline: 14.4423x; 1.0816x over previous
import functools

import jax
import jax.numpy as jnp
from jax import lax
from jax.experimental import pallas as pl
from jax.experimental.pallas import tpu as pltpu
from jax.experimental.pallas import tpu_sc as plsc

N = 10000
E = 160000
D = 128
Q = 32
NUM_PROP = 5
CAT_PROP = 3
DES = 768
TWEET = 768
KX = NUM_PROP + CAT_PROP + DES + TWEET

NC = 2
NS = 16
B = 128
CH = 80
NH = 2
HCH = CH // NH
EPAD = NS * CH * B
NPAD = N + 8
R0 = 624
TAIL = N - NS * R0
TB = N - TAIL


@functools.cache
def _sc_mesh():
    return plsc.VectorSubcoreMesh(
        core_axis_name="c", subcore_axis_name="s", num_cores=NC, num_subcores=NS
    )


def _sc_degree_body(ones_hbm, dst_hbm, out_hbm, dst_v, ones_v, acc):
    c = lax.axis_index("c")
    s = lax.axis_index("s")
    pltpu.sync_copy(ones_hbm, acc.at[pl.ds(s * R0, R0)])

    @pl.when(s == NS - 1)
    def _():
        pltpu.sync_copy(ones_hbm.at[pl.ds(0, TAIL)], acc.at[pl.ds(TB, TAIL)])

    pltpu.sync_copy(ones_hbm.at[pl.ds(0, B)], ones_v)
    plsc.subcore_barrier()

    def body(j, carry):
        pltpu.sync_copy(ones_v, acc.at[dst_v.at[j]], add=True)
        return carry

    for h in range(NH):
        pltpu.sync_copy(dst_hbm.at[c, s, h], dst_v)
        lax.fori_loop(0, HCH, body, 0)
    plsc.subcore_barrier()
    pltpu.sync_copy(
        acc.at[pl.ds(s * R0, R0)], out_hbm.at[c, pl.ds(s * R0, R0)])

    @pl.when(s == NS - 1)
    def _():
        pltpu.sync_copy(acc.at[pl.ds(TB, TAIL)], out_hbm.at[c, pl.ds(TB, TAIL)])


@functools.cache
def _sc_degree():
    return pl.kernel(
        _sc_degree_body,
        out_type=jax.ShapeDtypeStruct((NC, N, D), jnp.float32),
        mesh=_sc_mesh(),
        scratch_types=[
            pltpu.VMEM((HCH, B), jnp.int32),
            pltpu.VMEM((B, D), jnp.float32),
            pltpu.VMEM_SHARED((NPAD, D), jnp.float32),
        ],
    )


def _sc_aggregate_body(h_hbm, src_hbm, dst_hbm, out_hbm,
                       src_v, dst_v, rows0, rows1, acc, sem0, sem1):
    c = lax.axis_index("c")
    s = lax.axis_index("s")
    pltpu.sync_copy(
        h_hbm.at[pl.ds(c * N + s * R0, R0)], acc.at[pl.ds(s * R0, R0)])

    @pl.when(s == NS - 1)
    def _():
        pltpu.sync_copy(h_hbm.at[pl.ds(c * N + TB, TAIL)], acc.at[pl.ds(TB, TAIL)])

    plsc.subcore_barrier()

    def body(j, carry):
        pltpu.sync_copy(h_hbm.at[src_v.at[j]], rows0)
        pltpu.sync_copy(rows0, acc.at[dst_v.at[j]], add=True)
        return carry

    for h in range(NH):
        pltpu.sync_copy(src_hbm.at[c, s, h], src_v)
        pltpu.sync_copy(dst_hbm.at[c, s, h], dst_v)
        lax.fori_loop(0, HCH, body, 0)
    plsc.subcore_barrier()
    pltpu.sync_copy(
        acc.at[pl.ds(s * R0, R0)], out_hbm.at[c, pl.ds(s * R0, R0)])

    @pl.when(s == NS - 1)
    def _():
        pltpu.sync_copy(acc.at[pl.ds(TB, TAIL)], out_hbm.at[c, pl.ds(TB, TAIL)])


@functools.cache
def _sc_aggregate():
    return pl.kernel(
        _sc_aggregate_body,
        out_type=jax.ShapeDtypeStruct((NC, N, D), jnp.float32),
        mesh=_sc_mesh(),
        scratch_types=[
            pltpu.VMEM((HCH + 1, B), jnp.int32),
            pltpu.VMEM((HCH, B), jnp.int32),
            pltpu.VMEM((B, D), jnp.float32),
            pltpu.VMEM((B, D), jnp.float32),
            pltpu.VMEM_SHARED((NPAD, D), jnp.float32),
            pltpu.SemaphoreType.DMA,
            pltpu.SemaphoreType.DMA,
        ],
    )


def _lrelu(v):
    return jnp.where(v >= 0, v, 0.01 * v)


def _dot(a, b):
    return jnp.dot(a, b, preferred_element_type=jnp.float32)


BM = 400


def _tc_encode_body(x_ref, w1_ref, b1_ref, win_ref, bin_ref, emb_ref):
    feat = _lrelu(_dot(x_ref[...], w1_ref[...]) + b1_ref[...])
    emb_ref[...] = _lrelu(_dot(feat, win_ref[...]) + bin_ref[...])


def _tc_scale_body(emb_ref, cf_ref, cr_ref, wf_ref, wr_ref,
                   hs_ref, df_ref, dr_ref):
    emb = emb_ref[...]
    df = lax.rsqrt(cf_ref[...])
    dr = lax.rsqrt(cr_ref[...])
    df_ref[...] = df
    dr_ref[...] = dr
    hs_ref[...] = jnp.stack(
        [_dot(emb, wf_ref[...]) * df, _dot(emb, wr_ref[...]) * dr])


def _tc_mid_body(a_ref, df_ref, dr_ref, bf_ref, br_ref,
                 wf_ref, wr_ref, hs_ref):
    h1 = jnp.maximum(
        df_ref[...] * a_ref[0] + bf_ref[...]
        + dr_ref[...] * a_ref[1] + br_ref[...], 0.0)
    hs_ref[...] = jnp.stack(
        [_dot(h1, wf_ref[...]) * df_ref[...], _dot(h1, wr_ref[...]) * dr_ref[...]])


def _tc_out_body(a_ref, df_ref, dr_ref, bf_ref, br_ref, out_ref):
    out_ref[...] = jnp.maximum(
        df_ref[...] * a_ref[0] + bf_ref[...]
        + dr_ref[...] * a_ref[1] + br_ref[...], 0.0)


def _row_spec(w):
    return pl.BlockSpec((BM, w), lambda i: (i, 0))


def _stk_spec():
    return pl.BlockSpec((NC, BM, D), lambda i: (0, i, 0))


def _full_spec(h, w):
    return pl.BlockSpec((h, w), lambda i: (0, 0))


_HS = jax.ShapeDtypeStruct((NC, N, D), jnp.float32)

_tc_encode = pl.pallas_call(
    _tc_encode_body,
    grid=(N // BM,),
    in_specs=[
        _row_spec(KX),
        _full_spec(KX, D), _full_spec(1, D),
        _full_spec(D, D), _full_spec(1, D),
    ],
    out_specs=_row_spec(D),
    out_shape=jax.ShapeDtypeStruct((N, D), jnp.float32),
)

_tc_scale = pl.pallas_call(
    _tc_scale_body,
    grid=(N // BM,),
    in_specs=[
        _row_spec(D), _row_spec(1), _row_spec(1),
        _full_spec(D, D), _full_spec(D, D),
    ],
    out_specs=[_stk_spec(), _row_spec(1), _row_spec(1)],
    out_shape=[
        _HS,
        jax.ShapeDtypeStruct((N, 1), jnp.float32),
        jax.ShapeDtypeStruct((N, 1), jnp.float32),
    ],
)

_tc_mid = pl.pallas_call(
    _tc_mid_body,
    grid=(N // BM,),
    in_specs=[
        _stk_spec(), _row_spec(1), _row_spec(1),
        _full_spec(1, D), _full_spec(1, D),
        _full_spec(D, D), _full_spec(D, D),
    ],
    out_specs=_stk_spec(),
    out_shape=_HS,
)

_tc_out = pl.pallas_call(
    _tc_out_body,
    grid=(N // BM,),
    in_specs=[
        _stk_spec(), _row_spec(1), _row_spec(1),
        _full_spec(1, D), _full_spec(1, D),
    ],
    out_specs=_row_spec(D),
    out_shape=jax.ShapeDtypeStruct((N, D), jnp.float32),
)


def kernel(x_user, edge_index_follow, edge_index_friend,
           W_num, b_num, W_cat, b_cat, W_des, b_des, W_tweet, b_tweet,
           W_in, b_in, W_f0, b_f0, W_r0, b_r0, W_f1, b_f1, W_r1, b_r1):
    W1 = jnp.zeros((KX, D), jnp.float32)
    W1 = W1.at[:NUM_PROP, :Q].set(W_num)
    W1 = W1.at[NUM_PROP:NUM_PROP + CAT_PROP, Q:2 * Q].set(W_cat)
    W1 = W1.at[NUM_PROP + CAT_PROP:NUM_PROP + CAT_PROP + DES, 2 * Q:3 * Q].set(W_des)
    W1 = W1.at[NUM_PROP + CAT_PROP + DES:, 3 * Q:].set(W_tweet)
    b1 = jnp.concatenate([b_num, b_cat, b_des, b_tweet]).reshape(1, D)

    npad = EPAD - E
    pad8 = jnp.arange(npad, dtype=jnp.int32) % 8
    zchunk = jnp.zeros((NS, NH, 1, B), jnp.int32)

    def prep(ei, c):
        src = jnp.concatenate([ei[0] + c * N, pad8]).reshape(NS, NH, HCH, B)
        src = jnp.concatenate([src, zchunk], axis=2)
        dst = jnp.concatenate([ei[1], N + pad8]).reshape(NS, NH, HCH, B)
        return src, dst

    s_f, d_f = prep(edge_index_follow, 0)
    s_r, d_r = prep(edge_index_friend, 1)
    srcI = jnp.stack([s_f, s_r])
    dstI = jnp.stack([d_f, d_r])

    ones_r = jnp.ones((R0, D), jnp.float32)
    deg = _sc_degree()(ones_r, dstI)
    cnt_f = deg[0, :, 0:1]
    cnt_r = deg[1, :, 0:1]

    emb = _tc_encode(x_user, W1, b1, W_in, b_in.reshape(1, D))
    hs0, dinv_f, dinv_r = _tc_scale(emb, cnt_f, cnt_r, W_f0, W_r0)

    agg0 = _sc_aggregate()(hs0.reshape(NC * N, D), srcI, dstI)

    hs1 = _tc_mid(agg0, dinv_f, dinv_r,
                  b_f0.reshape(1, D), b_r0.reshape(1, D), W_f1, W_r1)

    agg1 = _sc_aggregate()(hs1.reshape(NC * N, D), srcI, dstI)

    return _tc_out(agg1, dinv_f, dinv_r,
                   b_f1.reshape(1, D), b_r1.reshape(1, D))


# speedup vs baseline: 16.8456x; 1.1664x over previous
import functools

import jax
import jax.numpy as jnp
from jax import lax
from jax.experimental import pallas as pl
from jax.experimental.pallas import tpu as pltpu
from jax.experimental.pallas import tpu_sc as plsc

N = 10000
E = 160000
D = 128
Q = 32
NUM_PROP = 5
CAT_PROP = 3
DES = 768
TWEET = 768
KX = NUM_PROP + CAT_PROP + DES + TWEET

NC = 2
NS = 16
B = 128
CH = 80
NH = 2
HCH = CH // NH
EPAD = NS * CH * B
NPAD = N + 8
R0 = 624
TAIL = N - NS * R0
TB = N - TAIL


@functools.cache
def _sc_mesh():
    return plsc.VectorSubcoreMesh(
        core_axis_name="c", subcore_axis_name="s", num_cores=NC, num_subcores=NS
    )


def _sc_degree_body(ones_hbm, dst_hbm, out_hbm, dst_v, ones_v, acc):
    c = lax.axis_index("c")
    s = lax.axis_index("s")
    pltpu.sync_copy(ones_hbm, acc.at[pl.ds(s * R0, R0)])

    @pl.when(s == NS - 1)
    def _():
        pltpu.sync_copy(ones_hbm.at[pl.ds(0, TAIL)], acc.at[pl.ds(TB, TAIL)])

    pltpu.sync_copy(ones_hbm.at[pl.ds(0, B)], ones_v)
    plsc.subcore_barrier()

    def body(j, carry):
        pltpu.sync_copy(ones_v, acc.at[dst_v.at[j]], add=True)
        return carry

    for h in range(NH):
        pltpu.sync_copy(dst_hbm.at[c, s, h], dst_v)
        lax.fori_loop(0, HCH, body, 0)
    plsc.subcore_barrier()
    pltpu.sync_copy(
        acc.at[pl.ds(s * R0, R0)], out_hbm.at[c, pl.ds(s * R0, R0)])

    @pl.when(s == NS - 1)
    def _():
        pltpu.sync_copy(acc.at[pl.ds(TB, TAIL)], out_hbm.at[c, pl.ds(TB, TAIL)])


@functools.cache
def _sc_degree():
    return pl.kernel(
        _sc_degree_body,
        out_type=jax.ShapeDtypeStruct((NC, N, D), jnp.float32),
        mesh=_sc_mesh(),
        scratch_types=[
            pltpu.VMEM((HCH, B), jnp.int32),
            pltpu.VMEM((B, D), jnp.float32),
            pltpu.VMEM_SHARED((NPAD, D), jnp.float32),
        ],
    )


def _sc_aggregate_body(h_hbm, src_hbm, dst_hbm, out_hbm,
                       src_v, dst_v, rows0, rows1, acc, sem0, sem1):
    c = lax.axis_index("c")
    s = lax.axis_index("s")
    pltpu.sync_copy(
        h_hbm.at[pl.ds(c * N + s * R0, R0)], acc.at[pl.ds(s * R0, R0)])

    @pl.when(s == NS - 1)
    def _():
        pltpu.sync_copy(h_hbm.at[pl.ds(c * N + TB, TAIL)], acc.at[pl.ds(TB, TAIL)])

    plsc.subcore_barrier()

    G = 8
    rows = (rows0, rows1)
    sems = (sem0, sem1)

    def group(g, carry):
        base = g * G
        d = [None, None]
        d[0] = pltpu.async_copy(h_hbm.at[src_v.at[base]], rows0, sem0)
        d[1] = pltpu.async_copy(h_hbm.at[src_v.at[base + 1]], rows1, sem1)
        for k in range(G):
            d[k % 2].wait()
            pltpu.sync_copy(rows[k % 2], acc.at[dst_v.at[base + k]], add=True)
            if k + 2 < G:
                d[k % 2] = pltpu.async_copy(
                    h_hbm.at[src_v.at[base + k + 2]], rows[k % 2], sems[k % 2])
        return carry

    for h in range(NH):
        pltpu.sync_copy(src_hbm.at[c, s, h], src_v)
        pltpu.sync_copy(dst_hbm.at[c, s, h], dst_v)
        lax.fori_loop(0, HCH // G, group, 0)
    plsc.subcore_barrier()
    pltpu.sync_copy(
        acc.at[pl.ds(s * R0, R0)], out_hbm.at[c, pl.ds(s * R0, R0)])

    @pl.when(s == NS - 1)
    def _():
        pltpu.sync_copy(acc.at[pl.ds(TB, TAIL)], out_hbm.at[c, pl.ds(TB, TAIL)])


@functools.cache
def _sc_aggregate():
    return pl.kernel(
        _sc_aggregate_body,
        out_type=jax.ShapeDtypeStruct((NC, N, D), jnp.float32),
        mesh=_sc_mesh(),
        scratch_types=[
            pltpu.VMEM((HCH + 1, B), jnp.int32),
            pltpu.VMEM((HCH, B), jnp.int32),
            pltpu.VMEM((B, D), jnp.float32),
            pltpu.VMEM((B, D), jnp.float32),
            pltpu.VMEM_SHARED((NPAD, D), jnp.float32),
            pltpu.SemaphoreType.DMA,
            pltpu.SemaphoreType.DMA,
        ],
    )


def _lrelu(v):
    return jnp.where(v >= 0, v, 0.01 * v)


def _dot(a, b):
    return jnp.dot(a, b, preferred_element_type=jnp.float32)


BM = 400


def _tc_encode_body(x_ref, w1_ref, b1_ref, win_ref, bin_ref, emb_ref):
    feat = _lrelu(_dot(x_ref[...], w1_ref[...]) + b1_ref[...])
    emb_ref[...] = _lrelu(_dot(feat, win_ref[...]) + bin_ref[...])


def _tc_scale_body(emb_ref, cf_ref, cr_ref, wf_ref, wr_ref,
                   hs_ref, df_ref, dr_ref):
    emb = emb_ref[...]
    df = lax.rsqrt(cf_ref[...])
    dr = lax.rsqrt(cr_ref[...])
    df_ref[...] = df
    dr_ref[...] = dr
    hs_ref[...] = jnp.stack(
        [_dot(emb, wf_ref[...]) * df, _dot(emb, wr_ref[...]) * dr])


def _tc_mid_body(a_ref, df_ref, dr_ref, bf_ref, br_ref,
                 wf_ref, wr_ref, hs_ref):
    h1 = jnp.maximum(
        df_ref[...] * a_ref[0] + bf_ref[...]
        + dr_ref[...] * a_ref[1] + br_ref[...], 0.0)
    hs_ref[...] = jnp.stack(
        [_dot(h1, wf_ref[...]) * df_ref[...], _dot(h1, wr_ref[...]) * dr_ref[...]])


def _tc_out_body(a_ref, df_ref, dr_ref, bf_ref, br_ref, out_ref):
    out_ref[...] = jnp.maximum(
        df_ref[...] * a_ref[0] + bf_ref[...]
        + dr_ref[...] * a_ref[1] + br_ref[...], 0.0)


def _row_spec(w):
    return pl.BlockSpec((BM, w), lambda i: (i, 0))


def _stk_spec():
    return pl.BlockSpec((NC, BM, D), lambda i: (0, i, 0))


def _full_spec(h, w):
    return pl.BlockSpec((h, w), lambda i: (0, 0))


_HS = jax.ShapeDtypeStruct((NC, N, D), jnp.float32)

_tc_encode = pl.pallas_call(
    _tc_encode_body,
    grid=(N // BM,),
    in_specs=[
        _row_spec(KX),
        _full_spec(KX, D), _full_spec(1, D),
        _full_spec(D, D), _full_spec(1, D),
    ],
    out_specs=_row_spec(D),
    out_shape=jax.ShapeDtypeStruct((N, D), jnp.float32),
)

_tc_scale = pl.pallas_call(
    _tc_scale_body,
    grid=(N // BM,),
    in_specs=[
        _row_spec(D), _row_spec(1), _row_spec(1),
        _full_spec(D, D), _full_spec(D, D),
    ],
    out_specs=[_stk_spec(), _row_spec(1), _row_spec(1)],
    out_shape=[
        _HS,
        jax.ShapeDtypeStruct((N, 1), jnp.float32),
        jax.ShapeDtypeStruct((N, 1), jnp.float32),
    ],
)

_tc_mid = pl.pallas_call(
    _tc_mid_body,
    grid=(N // BM,),
    in_specs=[
        _stk_spec(), _row_spec(1), _row_spec(1),
        _full_spec(1, D), _full_spec(1, D),
        _full_spec(D, D), _full_spec(D, D),
    ],
    out_specs=_stk_spec(),
    out_shape=_HS,
)

_tc_out = pl.pallas_call(
    _tc_out_body,
    grid=(N // BM,),
    in_specs=[
        _stk_spec(), _row_spec(1), _row_spec(1),
        _full_spec(1, D), _full_spec(1, D),
    ],
    out_specs=_row_spec(D),
    out_shape=jax.ShapeDtypeStruct((N, D), jnp.float32),
)


def kernel(x_user, edge_index_follow, edge_index_friend,
           W_num, b_num, W_cat, b_cat, W_des, b_des, W_tweet, b_tweet,
           W_in, b_in, W_f0, b_f0, W_r0, b_r0, W_f1, b_f1, W_r1, b_r1):
    W1 = jnp.zeros((KX, D), jnp.float32)
    W1 = W1.at[:NUM_PROP, :Q].set(W_num)
    W1 = W1.at[NUM_PROP:NUM_PROP + CAT_PROP, Q:2 * Q].set(W_cat)
    W1 = W1.at[NUM_PROP + CAT_PROP:NUM_PROP + CAT_PROP + DES, 2 * Q:3 * Q].set(W_des)
    W1 = W1.at[NUM_PROP + CAT_PROP + DES:, 3 * Q:].set(W_tweet)
    b1 = jnp.concatenate([b_num, b_cat, b_des, b_tweet]).reshape(1, D)

    npad = EPAD - E
    pad8 = jnp.arange(npad, dtype=jnp.int32) % 8
    zchunk = jnp.zeros((NS, NH, 1, B), jnp.int32)

    def prep(ei, c):
        src = jnp.concatenate([ei[0] + c * N, pad8]).reshape(NS, NH, HCH, B)
        src = jnp.concatenate([src, zchunk], axis=2)
        dst = jnp.concatenate([ei[1], N + pad8]).reshape(NS, NH, HCH, B)
        return src, dst

    s_f, d_f = prep(edge_index_follow, 0)
    s_r, d_r = prep(edge_index_friend, 1)
    srcI = jnp.stack([s_f, s_r])
    dstI = jnp.stack([d_f, d_r])

    ones_r = jnp.ones((R0, D), jnp.float32)
    deg = _sc_degree()(ones_r, dstI)
    cnt_f = deg[0, :, 0:1]
    cnt_r = deg[1, :, 0:1]

    emb = _tc_encode(x_user, W1, b1, W_in, b_in.reshape(1, D))
    hs0, dinv_f, dinv_r = _tc_scale(emb, cnt_f, cnt_r, W_f0, W_r0)

    agg0 = _sc_aggregate()(hs0.reshape(NC * N, D), srcI, dstI)

    hs1 = _tc_mid(agg0, dinv_f, dinv_r,
                  b_f0.reshape(1, D), b_r0.reshape(1, D), W_f1, W_r1)

    agg1 = _sc_aggregate()(hs1.reshape(NC * N, D), srcI, dstI)

    return _tc_out(agg1, dinv_f, dinv_r,
                   b_f1.reshape(1, D), b_r1.reshape(1, D))


# speedup vs baseline: 18.5656x; 1.1021x over previous
import functools

import jax
import jax.numpy as jnp
from jax import lax
from jax.experimental import pallas as pl
from jax.experimental.pallas import tpu as pltpu
from jax.experimental.pallas import tpu_sc as plsc

N = 10000
E = 160000
D = 128
Q = 32
NUM_PROP = 5
CAT_PROP = 3
DES = 768
TWEET = 768
KX = NUM_PROP + CAT_PROP + DES + TWEET

NC = 2
NS = 16
B = 128
CH = 80
NH = 2
HCH = CH // NH
EPAD = NS * CH * B
NPAD = N + 8
R0 = 624
TAIL = N - NS * R0
TB = N - TAIL


@functools.cache
def _sc_mesh():
    return plsc.VectorSubcoreMesh(
        core_axis_name="c", subcore_axis_name="s", num_cores=NC, num_subcores=NS
    )


def _sc_degree_body(ones_hbm, dst_hbm, out_hbm, dst_v, ones_v, acc):
    c = lax.axis_index("c")
    s = lax.axis_index("s")
    pltpu.sync_copy(ones_hbm, acc.at[pl.ds(s * R0, R0)])

    @pl.when(s == NS - 1)
    def _():
        pltpu.sync_copy(ones_hbm.at[pl.ds(0, TAIL)], acc.at[pl.ds(TB, TAIL)])

    pltpu.sync_copy(ones_hbm.at[pl.ds(0, B)], ones_v)
    plsc.subcore_barrier()

    def body(j, carry):
        pltpu.sync_copy(ones_v, acc.at[dst_v.at[j]], add=True)
        return carry

    for h in range(NH):
        pltpu.sync_copy(dst_hbm.at[c, s, h], dst_v)
        lax.fori_loop(0, HCH, body, 0)
    plsc.subcore_barrier()
    pltpu.sync_copy(
        acc.at[pl.ds(s * R0, R0)], out_hbm.at[c, pl.ds(s * R0, R0)])

    @pl.when(s == NS - 1)
    def _():
        pltpu.sync_copy(acc.at[pl.ds(TB, TAIL)], out_hbm.at[c, pl.ds(TB, TAIL)])


@functools.cache
def _sc_degree():
    return pl.kernel(
        _sc_degree_body,
        out_type=jax.ShapeDtypeStruct((NC, N, D), jnp.float32),
        mesh=_sc_mesh(),
        scratch_types=[
            pltpu.VMEM((HCH, B), jnp.int32),
            pltpu.VMEM((B, D), jnp.float32),
            pltpu.VMEM_SHARED((NPAD, D), jnp.float32),
        ],
    )


def _sc_aggregate_body(h_hbm, src_hbm, dst_hbm, out_hbm,
                       src_v, dst_v, rows0, rows1, acc, sem0, sem1):
    c = lax.axis_index("c")
    s = lax.axis_index("s")
    pltpu.sync_copy(
        h_hbm.at[pl.ds(c * N + s * R0, R0)], acc.at[pl.ds(s * R0, R0)])

    @pl.when(s == NS - 1)
    def _():
        pltpu.sync_copy(h_hbm.at[pl.ds(c * N + TB, TAIL)], acc.at[pl.ds(TB, TAIL)])

    plsc.subcore_barrier()

    G = 10
    rows = (rows0, rows1)
    sems = (sem0, sem1)

    def group(g, carry):
        base = g * G
        d = [None, None]
        d[0] = pltpu.async_copy(h_hbm.at[src_v.at[base]], rows0, sem0)
        d[1] = pltpu.async_copy(h_hbm.at[src_v.at[base + 1]], rows1, sem1)
        for k in range(G):
            d[k % 2].wait()
            pltpu.sync_copy(rows[k % 2], acc.at[dst_v.at[base + k]], add=True)
            if k + 2 < G:
                d[k % 2] = pltpu.async_copy(
                    h_hbm.at[src_v.at[base + k + 2]], rows[k % 2], sems[k % 2])
        return carry

    for h in range(NH):
        pltpu.sync_copy(src_hbm.at[c, s, h], src_v)
        pltpu.sync_copy(dst_hbm.at[c, s, h], dst_v)
        lax.fori_loop(0, HCH // G, group, 0)
    plsc.subcore_barrier()
    pltpu.sync_copy(
        acc.at[pl.ds(s * R0, R0)], out_hbm.at[c, pl.ds(s * R0, R0)])

    @pl.when(s == NS - 1)
    def _():
        pltpu.sync_copy(acc.at[pl.ds(TB, TAIL)], out_hbm.at[c, pl.ds(TB, TAIL)])


@functools.cache
def _sc_aggregate():
    return pl.kernel(
        _sc_aggregate_body,
        out_type=jax.ShapeDtypeStruct((NC, N, D), jnp.float32),
        mesh=_sc_mesh(),
        scratch_types=[
            pltpu.VMEM((HCH + 1, B), jnp.int32),
            pltpu.VMEM((HCH, B), jnp.int32),
            pltpu.VMEM((B, D), jnp.float32),
            pltpu.VMEM((B, D), jnp.float32),
            pltpu.VMEM_SHARED((NPAD, D), jnp.float32),
            pltpu.SemaphoreType.DMA,
            pltpu.SemaphoreType.DMA,
        ],
    )


def _lrelu(v):
    return jnp.where(v >= 0, v, 0.01 * v)


def _dot(a, b):
    return jnp.dot(a, b, preferred_element_type=jnp.float32)


BM = 400


def _tc_encode_body(xt_ref, w1_ref, b1_ref, win_ref, bin_ref, emb_ref):
    feat = _lrelu(
        lax.dot_general(xt_ref[...], w1_ref[...], (((0,), (0,)), ((), ())),
                        preferred_element_type=jnp.float32) + b1_ref[...])
    emb_ref[...] = _lrelu(_dot(feat, win_ref[...]) + bin_ref[...])


def _tc_scale_body(emb_ref, deg_ref, wf_ref, wr_ref,
                   hs_ref, df_ref, dr_ref):
    emb = emb_ref[...]
    df = lax.rsqrt(deg_ref[0, :, 0:1])
    dr = lax.rsqrt(deg_ref[1, :, 0:1])
    df_ref[...] = df
    dr_ref[...] = dr
    hs_ref[...] = jnp.stack(
        [_dot(emb, wf_ref[...]) * df, _dot(emb, wr_ref[...]) * dr])


def _tc_mid_body(a_ref, df_ref, dr_ref, bf_ref, br_ref,
                 wf_ref, wr_ref, hs_ref):
    h1 = jnp.maximum(
        df_ref[...] * a_ref[0] + bf_ref[...]
        + dr_ref[...] * a_ref[1] + br_ref[...], 0.0)
    hs_ref[...] = jnp.stack(
        [_dot(h1, wf_ref[...]) * df_ref[...], _dot(h1, wr_ref[...]) * dr_ref[...]])


def _tc_out_body(a_ref, df_ref, dr_ref, bf_ref, br_ref, out_ref):
    out_ref[...] = jnp.maximum(
        df_ref[...] * a_ref[0] + bf_ref[...]
        + dr_ref[...] * a_ref[1] + br_ref[...], 0.0)


def _row_spec(w):
    return pl.BlockSpec((BM, w), lambda i: (i, 0))


def _stk_spec():
    return pl.BlockSpec((NC, BM, D), lambda i: (0, i, 0))


def _full_spec(h, w):
    return pl.BlockSpec((h, w), lambda i: (0, 0))


_HS = jax.ShapeDtypeStruct((NC, N, D), jnp.float32)

BME = 512

_tc_encode = pl.pallas_call(
    _tc_encode_body,
    grid=(pl.cdiv(N, BME),),
    in_specs=[
        pl.BlockSpec((KX, BME), lambda i: (0, i)),
        _full_spec(KX, D), _full_spec(1, D),
        _full_spec(D, D), _full_spec(1, D),
    ],
    out_specs=pl.BlockSpec((BME, D), lambda i: (i, 0)),
    out_shape=jax.ShapeDtypeStruct((N, D), jnp.float32),
)

_tc_scale = pl.pallas_call(
    _tc_scale_body,
    grid=(N // BM,),
    in_specs=[
        _row_spec(D), _stk_spec(),
        _full_spec(D, D), _full_spec(D, D),
    ],
    out_specs=[_stk_spec(), _row_spec(1), _row_spec(1)],
    out_shape=[
        _HS,
        jax.ShapeDtypeStruct((N, 1), jnp.float32),
        jax.ShapeDtypeStruct((N, 1), jnp.float32),
    ],
)

_tc_mid = pl.pallas_call(
    _tc_mid_body,
    grid=(N // BM,),
    in_specs=[
        _stk_spec(), _row_spec(1), _row_spec(1),
        _full_spec(1, D), _full_spec(1, D),
        _full_spec(D, D), _full_spec(D, D),
    ],
    out_specs=_stk_spec(),
    out_shape=_HS,
)

_tc_out = pl.pallas_call(
    _tc_out_body,
    grid=(N // BM,),
    in_specs=[
        _stk_spec(), _row_spec(1), _row_spec(1),
        _full_spec(1, D), _full_spec(1, D),
    ],
    out_specs=_row_spec(D),
    out_shape=jax.ShapeDtypeStruct((N, D), jnp.float32),
)


def kernel(x_user, edge_index_follow, edge_index_friend,
           W_num, b_num, W_cat, b_cat, W_des, b_des, W_tweet, b_tweet,
           W_in, b_in, W_f0, b_f0, W_r0, b_r0, W_f1, b_f1, W_r1, b_r1):
    W1 = jnp.zeros((KX, D), jnp.float32)
    W1 = W1.at[:NUM_PROP, :Q].set(W_num)
    W1 = W1.at[NUM_PROP:NUM_PROP + CAT_PROP, Q:2 * Q].set(W_cat)
    W1 = W1.at[NUM_PROP + CAT_PROP:NUM_PROP + CAT_PROP + DES, 2 * Q:3 * Q].set(W_des)
    W1 = W1.at[NUM_PROP + CAT_PROP + DES:, 3 * Q:].set(W_tweet)
    b1 = jnp.concatenate([b_num, b_cat, b_des, b_tweet]).reshape(1, D)

    npad = EPAD - E
    pad8 = jnp.arange(npad, dtype=jnp.int32) % 8
    zchunk = jnp.zeros((NS, NH, 1, B), jnp.int32)

    def prep(ei, c):
        src = jnp.concatenate([ei[0] + c * N, pad8]).reshape(NS, NH, HCH, B)
        src = jnp.concatenate([src, zchunk], axis=2)
        dst = jnp.concatenate([ei[1], N + pad8]).reshape(NS, NH, HCH, B)
        return src, dst

    s_f, d_f = prep(edge_index_follow, 0)
    s_r, d_r = prep(edge_index_friend, 1)
    srcI = jnp.stack([s_f, s_r])
    dstI = jnp.stack([d_f, d_r])

    ones_r = jnp.ones((R0, D), jnp.float32)
    deg = _sc_degree()(ones_r, dstI)

    emb = _tc_encode(x_user.T, W1, b1, W_in, b_in.reshape(1, D))
    hs0, dinv_f, dinv_r = _tc_scale(emb, deg, W_f0, W_r0)

    agg0 = _sc_aggregate()(hs0.reshape(NC * N, D), srcI, dstI)

    hs1 = _tc_mid(agg0, dinv_f, dinv_r,
                  b_f0.reshape(1, D), b_r0.reshape(1, D), W_f1, W_r1)

    agg1 = _sc_aggregate()(hs1.reshape(NC * N, D), srcI, dstI)

    return _tc_out(agg1, dinv_f, dinv_r,
                   b_f1.reshape(1, D), b_r1.reshape(1, D))
